# Initial kernel scaffold; baseline (speedup 1.0000x reference)
import jax
import jax.numpy as jnp
from jax.experimental import pallas as pl


def kernel(features, central_species, W1, b1, W2, b2):
    raise NotImplementedError("write your pallas kernel here")


# speedup vs baseline: 1.8507x; 1.8507x over previous
import functools

import jax
import jax.numpy as jnp
from jax import lax
from jax.experimental import pallas as pl
from jax.experimental.pallas import tpu as pltpu
from jax.experimental.pallas import tpu_sc as plsc

BT = 256
FT = 1024
NW = 32
CH = 64


def _sc_gather_rows(table, idx):
    V, D = table.shape
    B = idx.shape[0]
    b_per_w = B // NW
    nch = b_per_w // CH
    mesh = plsc.VectorSubcoreMesh(core_axis_name="c", subcore_axis_name="s")

    @functools.partial(
        pl.kernel,
        mesh=mesh,
        out_type=jax.ShapeDtypeStruct((B, D), jnp.float32),
        scratch_types=[
            pltpu.VMEM((CH,), jnp.int32),
            pltpu.VMEM((CH, D), jnp.float32),
            pltpu.SemaphoreType.DMA,
        ],
    )
    def k(table_hbm, idx_hbm, out_hbm, idx_v, rows_v, sem):
        wid = lax.axis_index("s") * 2 + lax.axis_index("c")
        base = wid * b_per_w
        for c in range(nch):
            off = base + c * CH
            pltpu.sync_copy(idx_hbm.at[pl.ds(off, CH)], idx_v)
            pltpu.async_copy(table_hbm.at[idx_v], rows_v, sem).wait()
            pltpu.sync_copy(rows_v, out_hbm.at[pl.ds(off, CH)])

    return k(table, idx)


def _group_mlp(xp, W1, b1, W2, b2, block_expert, block_active, interpret=False):
    NP, D = xp.shape
    E, _, F = W1.shape
    NB = NP // BT
    NF = F // FT

    def body(be_ref, act_ref, x_ref, w1_ref, b1_ref, w2_ref, b2_ref,
             out_ref, acc_ref):
        f = pl.program_id(0)
        g = pl.program_id(1)
        row0 = g * BT

        @pl.when(act_ref[g] != 0)
        def _():
            x = x_ref[...]
            h = jnp.maximum(
                jnp.dot(x, w1_ref[0], preferred_element_type=jnp.float32)
                + b1_ref[0], 0.0)
            y = jnp.dot(h, w2_ref[0], preferred_element_type=jnp.float32)

            @pl.when(f == 0)
            def _():
                acc_ref[pl.ds(row0, BT), :] = y + b2_ref[0]

            @pl.when(f != 0)
            def _():
                acc_ref[pl.ds(row0, BT), :] = acc_ref[pl.ds(row0, BT), :] + y

        @pl.when(f == NF - 1)
        def _():
            out_ref[...] = acc_ref[pl.ds(row0, BT), :]

    grid_spec = pltpu.PrefetchScalarGridSpec(
        num_scalar_prefetch=2,
        grid=(NF, NB),
        in_specs=[
            pl.BlockSpec((BT, D), lambda f, g, be, act: (g, 0)),
            pl.BlockSpec((1, D, FT), lambda f, g, be, act: (be[g], 0, f)),
            pl.BlockSpec((1, 1, FT), lambda f, g, be, act: (be[g], 0, f)),
            pl.BlockSpec((1, FT, D), lambda f, g, be, act: (be[g], f, 0)),
            pl.BlockSpec((1, 1, D), lambda f, g, be, act: (be[g], 0, 0)),
        ],
        out_specs=pl.BlockSpec((BT, D), lambda f, g, be, act: (g, 0)),
        scratch_shapes=[pltpu.VMEM((NP, D), jnp.float32)],
    )
    return pl.pallas_call(
        body,
        grid_spec=grid_spec,
        out_shape=jax.ShapeDtypeStruct((NP, D), jnp.float32),
        interpret=interpret,
    )(block_expert, block_active, xp, W1,
      b1.reshape(E, 1, F), W2, b2.reshape(E, 1, D))


def _routing(central_species, N, E, NP, NB):
    sp = central_species.astype(jnp.int32)
    onehot = (sp[:, None] == jnp.arange(E, dtype=jnp.int32)[None, :])
    onehot = onehot.astype(jnp.int32)
    cnt = jnp.sum(onehot, axis=0)
    rank = jnp.cumsum(onehot, axis=0) - onehot
    rank_i = jnp.take_along_axis(rank, sp[:, None], axis=1)[:, 0]
    padded_cnt = ((cnt + BT - 1) // BT) * BT
    padded_off = jnp.concatenate(
        [jnp.zeros((1,), jnp.int32), jnp.cumsum(padded_cnt)[:-1]])
    dst = (padded_off[sp] + rank_i).astype(jnp.int32)
    gather_idx = jnp.zeros((NP,), jnp.int32).at[dst].set(
        jnp.arange(N, dtype=jnp.int32))
    nblk = padded_cnt // BT
    cum_blk = jnp.cumsum(nblk)
    gids = jnp.arange(NB, dtype=jnp.int32)
    be = jnp.searchsorted(cum_blk, gids, side="right").astype(jnp.int32)
    active = (gids < cum_blk[-1]).astype(jnp.int32)
    last_e = jnp.max(jnp.where(nblk > 0, jnp.arange(E, dtype=jnp.int32), -1))
    be = jnp.where(active != 0, jnp.minimum(be, E - 1), last_e)
    return gather_idx, dst, be, active


def kernel(features, central_species, W1, b1, W2, b2):
    N, D = features.shape
    E, _, F = W1.shape
    NP = N + E * BT
    NB = NP // BT
    gather_idx, dst, block_expert, block_active = _routing(
        central_species, N, E, NP, NB)
    xp = _sc_gather_rows(features, gather_idx)
    yp = _group_mlp(xp, W1, b1, W2, b2, block_expert, block_active)
    return _sc_gather_rows(yp, dst)


# speedup vs baseline: 2.4416x; 1.3193x over previous
import functools

import jax
import jax.numpy as jnp
from jax import lax
from jax.experimental import pallas as pl
from jax.experimental.pallas import tpu as pltpu
from jax.experimental.pallas import tpu_sc as plsc

BT = 256
FT = 1024
NW = 32
CH = 64


def _sc_gather_rows(table, idx):
    V, D = table.shape
    B = idx.shape[0]
    b_per_w = B // NW
    nch = b_per_w // CH
    mesh = plsc.VectorSubcoreMesh(core_axis_name="c", subcore_axis_name="s")

    @functools.partial(
        pl.kernel,
        mesh=mesh,
        out_type=jax.ShapeDtypeStruct((B, D), jnp.float32),
        scratch_types=[
            pltpu.VMEM((CH,), jnp.int32),
            pltpu.VMEM((CH, D), jnp.float32),
            pltpu.SemaphoreType.DMA,
        ],
    )
    def k(table_hbm, idx_hbm, out_hbm, idx_v, rows_v, sem):
        wid = lax.axis_index("s") * 2 + lax.axis_index("c")
        base = wid * b_per_w
        for c in range(nch):
            off = base + c * CH
            pltpu.sync_copy(idx_hbm.at[pl.ds(off, CH)], idx_v)
            pltpu.async_copy(table_hbm.at[idx_v], rows_v, sem).wait()
            pltpu.sync_copy(rows_v, out_hbm.at[pl.ds(off, CH)])

    return k(table, idx)


def _group_mlp(xp, W1, b1, W2, b2, block_expert, block_active, interpret=False):
    NP, D = xp.shape
    E, _, F = W1.shape
    NB = NP // BT
    NF = F // FT

    def body(be_ref, act_ref, x_ref, w1_ref, b1_ref, w2_ref, b2_ref,
             out_ref, acc_ref):
        f = pl.program_id(0)
        g = pl.program_id(1)
        row0 = g * BT

        @pl.when(act_ref[g] != 0)
        def _():
            x = x_ref[...]
            h = jnp.maximum(
                jnp.dot(x, w1_ref[0], preferred_element_type=jnp.float32)
                + b1_ref[0], 0.0)
            y = jnp.dot(h, w2_ref[0], preferred_element_type=jnp.float32)

            @pl.when(f == 0)
            def _():
                acc_ref[pl.ds(row0, BT), :] = y + b2_ref[0]

            @pl.when(f != 0)
            def _():
                acc_ref[pl.ds(row0, BT), :] = acc_ref[pl.ds(row0, BT), :] + y

        @pl.when(f == NF - 1)
        def _():
            out_ref[...] = acc_ref[pl.ds(row0, BT), :]

    grid_spec = pltpu.PrefetchScalarGridSpec(
        num_scalar_prefetch=2,
        grid=(NF, NB),
        in_specs=[
            pl.BlockSpec((BT, D), lambda f, g, be, act: (g, 0)),
            pl.BlockSpec((1, D, FT), lambda f, g, be, act: (be[g], 0, f)),
            pl.BlockSpec((1, 1, FT), lambda f, g, be, act: (be[g], 0, f)),
            pl.BlockSpec((1, FT, D), lambda f, g, be, act: (be[g], f, 0)),
            pl.BlockSpec((1, 1, D), lambda f, g, be, act: (be[g], 0, 0)),
        ],
        out_specs=pl.BlockSpec((BT, D), lambda f, g, be, act: (g, 0)),
        scratch_shapes=[pltpu.VMEM((NP, D), jnp.float32)],
    )
    return pl.pallas_call(
        body,
        grid_spec=grid_spec,
        out_shape=jax.ShapeDtypeStruct((NP, D), jnp.float32),
        interpret=interpret,
    )(block_expert, block_active, xp, W1,
      b1.reshape(E, 1, F), W2, b2.reshape(E, 1, D))


def _routing(central_species, N, E, NP, NB):
    sp = central_species.astype(jnp.int32)
    onehot = (sp[:, None] == jnp.arange(E, dtype=jnp.int32)[None, :])
    onehot = onehot.astype(jnp.int32)
    cnt = jnp.sum(onehot, axis=0)
    rank = jnp.cumsum(onehot, axis=0) - onehot
    rank_i = jnp.take_along_axis(rank, sp[:, None], axis=1)[:, 0]
    padded_cnt = ((cnt + BT - 1) // BT) * BT
    padded_off = jnp.concatenate(
        [jnp.zeros((1,), jnp.int32), jnp.cumsum(padded_cnt)[:-1]])
    dst = (padded_off[sp] + rank_i).astype(jnp.int32)
    gather_idx = (jnp.arange(NP, dtype=jnp.int32) % N).at[dst].set(
        jnp.arange(N, dtype=jnp.int32))
    nblk = padded_cnt // BT
    cum_blk = jnp.cumsum(nblk)
    gids = jnp.arange(NB, dtype=jnp.int32)
    be = jnp.searchsorted(cum_blk, gids, side="right").astype(jnp.int32)
    active = (gids < cum_blk[-1]).astype(jnp.int32)
    last_e = jnp.max(jnp.where(nblk > 0, jnp.arange(E, dtype=jnp.int32), -1))
    be = jnp.where(active != 0, jnp.minimum(be, E - 1), last_e)
    return gather_idx, dst, be, active


def kernel(features, central_species, W1, b1, W2, b2):
    N, D = features.shape
    E, _, F = W1.shape
    NP = N + E * BT
    NB = NP // BT
    gather_idx, dst, block_expert, block_active = _routing(
        central_species, N, E, NP, NB)
    xp = _sc_gather_rows(features, gather_idx)
    yp = _group_mlp(xp, W1, b1, W2, b2, block_expert, block_active)
    return _sc_gather_rows(yp, dst)
